# baseline (speedup 1.0000x reference)
import functools

import jax
import jax.numpy as jnp
from jax import lax
from jax.experimental import pallas as pl
from jax.experimental.pallas import tpu as pltpu
from jax.experimental.pallas import tpu_sc as plsc

_ALPHA = 0.7
_NC, _NS, _L = 2, 16, 16
_LN2 = 0.6931471805599453


def _ln_clamped(x):
    bits = lax.bitcast_convert_type(x, jnp.int32)
    e = (bits >> 23) - 127
    m = lax.bitcast_convert_type(
        (bits & 0x007FFFFF) | 0x3F800000, jnp.float32)
    t = (m - 1.0) / (m + 1.0)
    t2 = t * t
    ln_m = t * (2.0 + t2 * (2.0 / 3.0 + t2 * (0.4 + t2 * (2.0 / 7.0
                                                          + t2 * (2.0 / 9.0)))))
    ln = e.astype(jnp.float32) * _LN2 + ln_m
    return jnp.where(x == 0.0, -100.0, jnp.maximum(ln, -100.0))


def _fused_sc(table, tgt_p, mask_p, pw_p, tok_p, n, v):
    per_w = n // _NS
    n_sub = per_w // 128
    chunks = 128 // _L
    t2sz = n // 4
    mesh = plsc.VectorSubcoreMesh(core_axis_name="c", subcore_axis_name="s")

    @functools.partial(
        pl.kernel,
        mesh=mesh,
        out_type=jax.ShapeDtypeStruct((_NC, 128), jnp.float32),
        compiler_params=pltpu.CompilerParams(needs_layout_passes=False),
        scratch_types=[
            pltpu.VMEM((per_w,), jnp.int32),
            pltpu.VMEM((per_w,), jnp.float32),
            pltpu.VMEM((per_w,), jnp.float32),
            pltpu.VMEM((per_w,), jnp.float32),
            pltpu.VMEM((n_sub, 128), jnp.int32),
            pltpu.VMEM((per_w,), jnp.float32),
            pltpu.VMEM((64,), jnp.float32),
            pltpu.VMEM_SHARED((_NS, 64), jnp.float32),
            pltpu.VMEM((_NS, 64), jnp.float32),
            pltpu.VMEM((48,), jnp.float32),
            pltpu.VMEM((128,), jnp.float32),
            pltpu.SemaphoreType.DMA,
            pltpu.SemaphoreType.DMA,
        ],
    )
    def k(table_hbm, tgt_hbm, mask_hbm, pw_hbm, tok_hbm, out_hbm,
          tgt_v, m_v, p_v, tk_v, idx_v, g_v, part_v, shared, red_v, sum_v,
          fin_v, sem, gsem):
        cid = lax.axis_index("c")
        sid = lax.axis_index("s")
        base = sid * per_w
        boff = (base // t2sz) * 128
        o0 = ((base % t2sz) >> 7) * 512 + boff
        cps = [pltpu.async_copy(tgt_hbm.at[pl.ds(o0 + kk * 512, 128)],
                                tgt_v.at[pl.ds(kk * 128, 128)], sem)
               for kk in range(n_sub)]
        lin = []
        for hbm, vmem in ((mask_hbm, m_v), (pw_hbm, p_v), (tok_hbm, tk_v)):
            for kk in range(n_sub):
                lin.append(pltpu.async_copy(
                    hbm.at[pl.ds(o0 + kk * 512, 128)],
                    vmem.at[pl.ds(kk * 128, 128)], sem))
        for cp in cps:
            cp.wait()
        gcps = []
        for sb in range(n_sub):
            for c in range(chunks):
                off = sb * 128 + c * _L
                t = tgt_v[pl.ds(off, _L)]
                gi = base + off + lax.iota(jnp.int32, _L)
                idx_v[sb, pl.ds(c * _L, _L)] = (
                    (gi >> 3) * (v * 8) + ((gi & 7) << 7)
                    + ((t >> 7) << 10) + (t & 127))
            gcps.append(pltpu.async_copy(table_hbm.at[idx_v.at[sb]],
                                         g_v.at[pl.ds(sb * 128, 128)], gsem))
        for cp in lin:
            cp.wait()
        for cp in gcps:
            cp.wait()
        acc1 = jnp.zeros((_L,), jnp.float32)
        acc2 = jnp.zeros((_L,), jnp.float32)
        acc3 = jnp.zeros((_L,), jnp.float32)
        for c in range(per_w // _L):
            sl = pl.ds(c * _L, _L)
            g = g_v[sl]
            m = m_v[sl]
            p = p_v[sl]
            tk = tk_v[sl]
            acc1 = acc1 + g * m
            acc2 = acc2 + jnp.where(m > 0.0, 1.0, 0.0)
            acc3 = acc3 + tk * _ln_clamped(p) + (1.0 - tk) * _ln_clamped(1.0 - p)
        part_v[pl.ds(0, _L)] = acc1
        part_v[pl.ds(16, _L)] = acc2
        part_v[pl.ds(32, _L)] = acc3
        part_v[pl.ds(48, _L)] = jnp.zeros((_L,), jnp.float32)
        pltpu.sync_copy(part_v, shared.at[sid])
        plsc.subcore_barrier()

        @pl.when(sid == 0)
        def _():
            pltpu.sync_copy(shared, red_v)
            r1 = jnp.zeros((_L,), jnp.float32)
            r2 = jnp.zeros((_L,), jnp.float32)
            r3 = jnp.zeros((_L,), jnp.float32)
            for r in range(_NS):
                r1 = r1 + red_v[r, pl.ds(0, _L)]
                r2 = r2 + red_v[r, pl.ds(16, _L)]
                r3 = r3 + red_v[r, pl.ds(32, _L)]
            z16 = jnp.zeros((_L,), jnp.int32)
            for z in range(3):
                sum_v[pl.ds(z * _L, _L)] = jnp.zeros((_L,), jnp.float32)
            plsc.addupdate_scatter(sum_v, [z16], r1)
            plsc.addupdate_scatter(sum_v, [z16 + 16], r2)
            plsc.addupdate_scatter(sum_v, [z16 + 32], r3)
            v1 = plsc.load_gather(sum_v, [z16])
            v2 = plsc.load_gather(sum_v, [z16 + 16])
            v3 = plsc.load_gather(sum_v, [z16 + 32])
            vn = jnp.full((_L,), float(n), jnp.float32)
            fin_v[pl.ds(0, _L)] = ((-v1 / v2) * _ALPHA
                                   + (-v3 / vn) * (1.0 - _ALPHA))
            zero = jnp.zeros((_L,), jnp.float32)
            for z in range(1, 8):
                fin_v[pl.ds(z * _L, _L)] = zero
            pltpu.sync_copy(fin_v, out_hbm.at[cid])

    return k(table, tgt_p, mask_p, pw_p, tok_p)


def _perm_flat(x):
    four, t2 = x.shape
    return x.reshape(four, t2 // 128, 128).transpose(1, 0, 2).reshape(-1)


def kernel(inputs, target, mask, prob_w, token):
    _, B, T, V = inputs.shape
    n = B * T
    table = (inputs.reshape(n // 8, 8, V // 128, 128)
             .transpose(0, 2, 1, 3).reshape(n * V))
    out = _fused_sc(table, _perm_flat(target), _perm_flat(mask),
                    _perm_flat(prob_w), _perm_flat(token), n, V)
    return out[0, 0].reshape(())
